# baseline (speedup 1.0000x reference)
import functools

import jax
import jax.numpy as jnp
from jax import lax
from jax.experimental import pallas as pl
from jax.experimental.pallas import tpu as pltpu
from jax.experimental.pallas import tpu_sc as plsc

_VOCAB = 50257
_N_POS = 1024
_D = 768
_B = 16
_S = 1024
_N = _B * _S
_NC = 2
_NS = 16
_NW = _NC * _NS
_CW = _S // _NW
_LANES = _D // 16


def _make_emb_kernel():
  mesh = plsc.VectorSubcoreMesh(core_axis_name="c", subcore_axis_name="s")

  @functools.partial(
      pl.kernel,
      mesh=mesh,
      out_type=jax.ShapeDtypeStruct((_N, _D), jnp.float32),
      scratch_types=[
          pltpu.VMEM((_B, _CW), jnp.int32),
          pltpu.VMEM((_CW, _D), jnp.float32),
          pltpu.VMEM((_CW, _D), jnp.float32),
          pltpu.VMEM((_CW, _D), jnp.float32),
          pltpu.VMEM((_CW, _D), jnp.float32),
          pltpu.VMEM((_CW, _D), jnp.float32),
          pltpu.SemaphoreType.DMA,
          pltpu.SemaphoreType.DMA,
          pltpu.SemaphoreType.DMA,
          pltpu.SemaphoreType.DMA,
          pltpu.SemaphoreType.DMA,
          pltpu.SemaphoreType.DMA,
      ],
  )
  def emb(idx_hbm, tok_hbm, pos_hbm, out_hbm,
          idx_v, pos_v, gb0, gb1, ob0, ob1,
          semi, semp, sg0, sg1, so0, so1):
    wid = lax.axis_index("s") * _NC + lax.axis_index("c")
    col0 = wid * _CW
    his = [
        pltpu.async_copy(
            idx_hbm.at[pl.ds(b * _S + col0, _CW)], idx_v.at[b], semi)
        for b in range(_B)
    ]
    hp = pltpu.async_copy(pos_hbm.at[pl.ds(col0, _CW)], pos_v, semp)
    gbufs = [gb0, gb1]
    obufs = [ob0, ob1]
    sgs = [sg0, sg1]
    sos = [so0, so1]
    g = [None, None]
    o = [None, None]
    for h in his:
      h.wait()
    g[0] = pltpu.async_copy(tok_hbm.at[idx_v.at[0]], gbufs[0], sgs[0])
    g[1] = pltpu.async_copy(tok_hbm.at[idx_v.at[1]], gbufs[1], sgs[1])
    hp.wait()
    for b in range(_B):
      k = b & 1
      g[k].wait()
      if o[k] is not None:
        o[k].wait()
      gbuf = gbufs[k]
      obuf = obufs[k]

      def add_row(r, _, gbuf=gbuf, obuf=obuf):
        for j in range(_LANES):
          sl = pl.ds(j * 16, 16)
          obuf[r, sl] = gbuf[r, sl] + pos_v[r, sl]
        return ()

      lax.fori_loop(0, _CW, add_row, ())
      o[k] = pltpu.async_copy(
          obuf, out_hbm.at[pl.ds(b * _S + col0, _CW)], sos[k])
      if b + 2 < _B:
        g[k] = pltpu.async_copy(
            tok_hbm.at[idx_v.at[b + 2]], gbufs[k], sgs[k])
    o[0].wait()
    o[1].wait()

  return emb


_emb = _make_emb_kernel()


def kernel(input, token_table, pos_table):
  idx = input.reshape(_N).astype(jnp.int32)
  out = _emb(idx, token_table, pos_table)
  return out.reshape(_B, _S, _D)
